# Initial kernel scaffold; baseline (speedup 1.0000x reference)
import jax
import jax.numpy as jnp
from jax.experimental import pallas as pl


def kernel(gnnfeats, gnnpos, gnnfeats_batch, params):
    raise NotImplementedError("write your pallas kernel here")


# speedup vs baseline: 1.9220x; 1.9220x over previous
import functools

import jax
import jax.numpy as jnp
from jax.experimental import pallas as pl

_P = 20
_IN_FEAT = 14
_CONV_KS = (2, 4, 8, 16, 20)
_BN_EPS = 1e-5
_LN_EPS = 1e-5


def _fold_conv(p, cin):
    inv = 1.0 / jnp.sqrt(1.0 + _BN_EPS)
    return (
        p['W1'][:cin],
        p['W1'][cin:],
        (p['g1'] * inv).reshape(1, -1), p['be1'].reshape(1, -1),
        p['W2'], (p['g2'] * inv).reshape(1, -1), p['be2'].reshape(1, -1),
        p['W3'], (p['g3'] * inv).reshape(1, -1), p['be3'].reshape(1, -1),
        p['Ws'], (p['gs'] * inv).reshape(1, -1), p['bes'].reshape(1, -1),
    )


def _fold_ffn(p):
    out = []
    for g, b in p['ln']:
        out.append(g.reshape(1, -1))
        out.append(b.reshape(1, -1))
    for w, b in zip(p['W'], p['b']):
        out.append(w)
        out.append(b.reshape(1, -1))
    return tuple(out)


def _mm(x, w):
    return jax.lax.dot_general(
        x.astype(jnp.bfloat16), w.astype(jnp.bfloat16),
        (((1,), (0,)), ((), ())),
        preferred_element_type=jnp.float32)


def _ln(x, g, b):
    m = jnp.mean(x, axis=-1, keepdims=True)
    v = jnp.mean((x - m) ** 2, axis=-1, keepdims=True)
    return (x - m) * jax.lax.rsqrt(v + _LN_EPS) * g + b


def _elu(x):
    return jnp.where(x > 0, x, jnp.exp(jnp.minimum(x, 0.0)) - 1.0)


def _ffn_apply(w, x):
    ln = w[:12]
    wb = w[12:]
    x = _ln(x, ln[0], ln[1])
    for i in range(4):
        x = _elu(_mm(x, wb[2 * i]) + wb[2 * i + 1])
        x = _ln(x, ln[2 * i + 2], ln[2 * i + 3])
    x = _elu(x)
    x = _ln(x, ln[10], ln[11])
    return _mm(x, wb[8]) + wb[9]


def _conv_apply(w, fts, pts, keff, gb):
    w1a, w1b, s1, b1, w2, s2, b2, w3, s3, b3, ws, ss, bs = w
    cin = w1a.shape[0]
    c0 = w1a.shape[1]
    c2 = w3.shape[1]

    u = _mm(fts, w1a)
    fts3 = fts.reshape(gb, _P, cin)
    diff = (fts3[:, :, None, :] - fts3[:, None, :, :])
    v = _mm(diff.reshape(gb * _P * _P, cin), w1b)
    h1 = (v.reshape(gb, _P, _P, c0) + u.reshape(gb, 1, _P, c0)) * s1 + b1
    h1 = jnp.maximum(h1, 0.0).reshape(gb * _P * _P, c0)
    h2 = jnp.maximum(_mm(h1, w2) * s2 + b2, 0.0)
    h3 = (jnp.maximum(_mm(h2, w3) * s3 + b3, 0.0)
          .reshape(gb, _P, _P, c2))

    ii = jax.lax.broadcasted_iota(jnp.int32, (gb, _P, _P), 1)
    mm = jax.lax.broadcasted_iota(jnp.int32, (gb, _P, _P), 2)
    if keff < _P - 1:
        diff = pts[:, :, None, :] - pts[:, None, :, :]
        d3 = jnp.sum(diff * diff, axis=-1)
        d3 = d3 + jnp.where(ii == mm, jnp.float32(1e9), jnp.float32(0.0))

    aggr = jnp.full((gb, _P, c2), -1e30, dtype=jnp.float32)
    for j in range(_P):
        if keff < _P - 1:
            dj = d3[:, :, j:j + 1]
            less = (d3 < dj) | ((d3 == dj) & (mm < j))
            rank = jnp.sum(less.astype(jnp.float32), axis=-1,
                           keepdims=True)
            sel = rank < keff
        else:
            sel = ii[:, :, j:j + 1] != j
        hj = h3[:, j]
        aggr = jnp.maximum(aggr, jnp.where(sel, hj, -1e30))

    skip = _mm(fts, ws) * ss + bs
    return jnp.maximum(aggr.reshape(gb * _P, c2) + skip, 0.0)


def _conv_kernel_body(gb, feats_ref, pos_ref, *refs):
    it = iter(refs)
    conv_ws = [tuple(next(it)[...] for _ in range(13)) for _ in range(5)]
    in_s = next(it)[...]
    in_b = next(it)[...]
    o_fts, o_fromtau = (next(it) for _ in range(2))

    fts = feats_ref[...] * in_s + in_b
    pts = pos_ref[...].reshape(gb, _P, 3)

    for li in range(5):
        keff = min(_CONV_KS[li], _P - 1)
        fts = _conv_apply(conv_ws[li], fts, pts, keff, gb)
        pts = fts.reshape(gb, _P, fts.shape[1])

    o_fts[...] = fts
    o_fromtau[0] = jax.nn.sigmoid(jnp.max(fts.reshape(gb, _P, _P), axis=1))


def _head_kernel_body(nf_ref, *refs):
    it = iter(refs)
    enc_w = tuple(next(it)[...] for _ in range(22))
    istau_w = tuple(next(it)[...] for _ in range(22))
    p4_w = tuple(next(it)[...] for _ in range(22))
    dmode_w = tuple(next(it)[...] for _ in range(22))
    charge_w = tuple(next(it)[...] for _ in range(22))
    o_istau, o_p4, o_charge, o_dmode = (next(it) for _ in range(4))

    enc = _ffn_apply(enc_w, nf_ref[...])
    o_istau[...] = jax.nn.sigmoid(_ffn_apply(istau_w, enc))
    o_p4[...] = 200.0 * _ffn_apply(p4_w, enc)
    o_charge[...] = jax.nn.softmax(_ffn_apply(charge_w, enc), axis=-1)
    o_dmode[...] = jax.nn.softmax(_ffn_apply(dmode_w, enc), axis=-1)


def _full_spec(x):
    nd = x.ndim
    return pl.BlockSpec(x.shape, lambda i, _n=nd: (0,) * _n)


@functools.partial(jax.jit, static_argnames=('interpret',))
def _run(gnnfeats, gnnpos, params, interpret=False):
    g = gnnfeats.shape[0] // _P
    gb = 20
    grid = g // gb

    cins = (_IN_FEAT, 64, 256, 256, 256)
    conv_flat = []
    for li in range(5):
        conv_flat.extend(_fold_conv(params['convs'][li], cins[li]))
    conv_flat.append((params['in_g'] / jnp.sqrt(1.0 + _BN_EPS)).reshape(1, -1))
    conv_flat.append(params['in_b'].reshape(1, -1))

    conv_out_shapes = (
        jax.ShapeDtypeStruct((g * _P, _P), jnp.float32),
        jax.ShapeDtypeStruct((grid, gb, _P), jnp.float32),
    )
    conv_out_specs = (
        pl.BlockSpec((gb * _P, _P), lambda i: (i, 0)),
        pl.BlockSpec((1, gb, _P), lambda i: (i, 0, 0)),
    )
    conv_in_specs = [
        pl.BlockSpec((gb * _P, _IN_FEAT), lambda i: (i, 0)),
        pl.BlockSpec((gb * _P, 3), lambda i: (i, 0)),
    ] + [_full_spec(w) for w in conv_flat]

    fts, fromtau = pl.pallas_call(
        functools.partial(_conv_kernel_body, gb),
        grid=(grid,),
        in_specs=conv_in_specs,
        out_specs=conv_out_specs,
        out_shape=conv_out_shapes,
        interpret=interpret,
    )(gnnfeats, gnnpos, *conv_flat)

    nf = jnp.concatenate(
        [gnnfeats.reshape(g, _P, _IN_FEAT), fts.reshape(g, _P, _P)],
        axis=-1).reshape(g, _P * (_IN_FEAT + _P))

    head_flat = []
    for name in ('enc', 'istau', 'p4', 'dmode', 'charge'):
        head_flat.extend(_fold_ffn(params[name]))

    odims = (1, 4, 2, 6)
    head_out_shapes = tuple(
        jax.ShapeDtypeStruct((g, od), jnp.float32) for od in odims)
    head_out_specs = tuple(
        pl.BlockSpec((g, od), lambda i: (0, 0)) for od in odims)
    head_in_specs = [pl.BlockSpec((g, nf.shape[1]), lambda i: (0, 0))] + [
        _full_spec(w) for w in head_flat]

    istau, p4, charge, dmode = pl.pallas_call(
        _head_kernel_body,
        grid=(1,),
        in_specs=head_in_specs,
        out_specs=head_out_specs,
        out_shape=head_out_shapes,
        interpret=interpret,
    )(nf, *head_flat)

    return (fromtau.reshape(g, _P), istau, p4, charge, dmode)


def kernel(gnnfeats, gnnpos, gnnfeats_batch, params):
    del gnnfeats_batch
    return _run(gnnfeats, gnnpos, params)
